# baseline (speedup 1.0000x reference)
import functools

import jax
import jax.numpy as jnp
from jax import lax
from jax.experimental import pallas as pl
from jax.experimental.pallas import tpu as pltpu
from jax.experimental.pallas import tpu_sc as plsc

N_NODES = 10000
N_EDGES = 320000
D = 128
NC, NS = 2, 16
NW = NC * NS
E_W = N_EDGES // NW
CHUNK = 80
IDX_ROWS = E_W // CHUNK
ROW_BASE = 624
ROW_SPAN = 640


@functools.partial(
    pl.kernel,
    out_type=jax.ShapeDtypeStruct((NC, N_NODES, D), jnp.float32),
    mesh=plsc.VectorSubcoreMesh(core_axis_name="c", subcore_axis_name="s"),
    scratch_types=[
        pltpu.VMEM((IDX_ROWS, CHUNK), jnp.int32),
        pltpu.VMEM((CHUNK, D), jnp.float32),
        pltpu.VMEM((CHUNK, D), jnp.float32),
        pltpu.VMEM((CHUNK, D), jnp.float32),
        pltpu.VMEM_SHARED((N_NODES, D), jnp.float32),
        pltpu.SemaphoreType.DMA,
        pltpu.SemaphoreType.DMA,
        pltpu.SemaphoreType.DMA,
        pltpu.SemaphoreType.DMA,
        pltpu.SemaphoreType.DMA,
        pltpu.SemaphoreType.DMA,
    ],
)
def _seg_sum_sc(x_hbm, idx_hbm, partial_hbm, idx_v, xbuf0, xbuf1, xbuf2,
                acc, sg0, sg1, sg2, ss0, ss1, ss2):
    c = lax.axis_index("c")
    s = lax.axis_index("s")
    wid = s * NC + c

    with jax.named_scope("zero_acc"):
        zv = jnp.zeros((16,), jnp.float32)
        for i in range(CHUNK):
            for j in range(D // 16):
                xbuf0[i, pl.ds(j * 16, 16)] = zv
        for r in range(ROW_SPAN // CHUNK):
            pltpu.async_copy(
                xbuf0, acc.at[pl.ds(s * ROW_BASE + r * CHUNK, CHUNK)], sg0)
        pltpu.sync_copy(idx_hbm.at[wid], idx_v)
        for r in range(ROW_SPAN // CHUNK):
            pltpu.make_async_copy(
                xbuf0, acc.at[pl.ds(s * ROW_BASE, CHUNK)], sg0).wait()
        plsc.subcore_barrier()

    ebase = wid * E_W
    bufs = (xbuf0, xbuf1, xbuf2)
    gsems = (sg0, sg1, sg2)
    ssems = (ss0, ss1, ss2)

    def fire_g(j, b):
        pltpu.async_copy(
            x_hbm.at[pl.ds(ebase + j * CHUNK, CHUNK)], bufs[b], gsems[b])

    def drain_g(b):
        pltpu.make_async_copy(
            x_hbm.at[pl.ds(ebase, CHUNK)], bufs[b], gsems[b]).wait()

    def fire_s(j, b):
        pltpu.async_copy(bufs[b], acc.at[idx_v.at[j]], ssems[b], add=True)

    def drain_s(b):
        pltpu.make_async_copy(bufs[b], acc.at[idx_v.at[0]], ssems[b]).wait()

    fire_g(0, 0)
    drain_g(0)
    fire_s(0, 0)
    fire_g(1, 1)
    drain_g(1)
    fire_s(1, 1)
    fire_g(2, 2)

    def body(p, carry):
        k = 3 * p + 2
        drain_g(2)
        fire_s(k, 2)
        drain_s(0)
        fire_g(k + 1, 0)
        drain_g(0)
        fire_s(k + 1, 0)
        drain_s(1)
        fire_g(k + 2, 1)
        drain_g(1)
        fire_s(k + 2, 1)
        drain_s(2)
        fire_g(k + 3, 2)
        return carry

    lax.fori_loop(0, (IDX_ROWS - 5) // 3, body, 0)
    drain_g(2)
    fire_s(IDX_ROWS - 3, 2)
    drain_s(0)
    fire_g(IDX_ROWS - 2, 0)
    drain_g(0)
    fire_s(IDX_ROWS - 2, 0)
    drain_s(1)
    fire_g(IDX_ROWS - 1, 1)
    drain_g(1)
    fire_s(IDX_ROWS - 1, 1)
    drain_s(2)
    drain_s(0)
    drain_s(1)

    plsc.subcore_barrier()
    pltpu.sync_copy(
        acc.at[pl.ds(s * ROW_BASE, ROW_SPAN)],
        partial_hbm.at[c, pl.ds(s * ROW_BASE, ROW_SPAN)],
    )


def _combine_body(p_ref, o_ref):
    o_ref[...] = p_ref[0] + p_ref[1]


_N_BLK = 10


def _combine(partial):
    return pl.pallas_call(
        _combine_body,
        grid=(_N_BLK,),
        in_specs=[
            pl.BlockSpec((NC, N_NODES // _N_BLK, D), lambda i: (0, i, 0))
        ],
        out_specs=pl.BlockSpec((N_NODES // _N_BLK, D), lambda i: (i, 0)),
        out_shape=jax.ShapeDtypeStruct((N_NODES, D), jnp.float32),
    )(partial)


def kernel(x, index, dim_size):
    idx2d = index.astype(jnp.int32).reshape(NW, IDX_ROWS, CHUNK)
    partial = _seg_sum_sc(x, idx2d)
    return _combine(partial)


# speedup vs baseline: 1.2385x; 1.2385x over previous
import functools

import jax
import jax.numpy as jnp
from jax import lax
from jax.experimental import pallas as pl
from jax.experimental.pallas import tpu as pltpu
from jax.experimental.pallas import tpu_sc as plsc

N_NODES = 10000
N_EDGES = 320000
D = 128
NC, NS = 2, 16
NW = NC * NS
E_W = N_EDGES // NW
CHUNK = 80
IDX_ROWS = E_W // CHUNK
ROW_BASE = 624
ROW_SPAN = 640


@functools.partial(
    pl.kernel,
    out_type=jax.ShapeDtypeStruct((NC, N_NODES, D), jnp.float32),
    mesh=plsc.VectorSubcoreMesh(core_axis_name="c", subcore_axis_name="s"),
    scratch_types=[
        pltpu.VMEM((IDX_ROWS, CHUNK), jnp.int32),
        pltpu.VMEM((CHUNK, D), jnp.float32),
        pltpu.VMEM((CHUNK, D), jnp.float32),
        pltpu.VMEM_SHARED((N_NODES, D), jnp.float32),
        pltpu.SemaphoreType.DMA,
        pltpu.SemaphoreType.DMA,
    ],
)
def _seg_sum_sc(x_hbm, idx_hbm, partial_hbm, idx_v, xbuf0, xbuf1,
                acc, sg0, sg1):
    c = lax.axis_index("c")
    s = lax.axis_index("s")
    wid = s * NC + c

    with jax.named_scope("zero_acc"):
        zv = jnp.zeros((16,), jnp.float32)
        for i in range(CHUNK):
            for j in range(D // 16):
                xbuf0[i, pl.ds(j * 16, 16)] = zv
        for r in range(ROW_SPAN // CHUNK):
            pltpu.async_copy(
                xbuf0, acc.at[pl.ds(s * ROW_BASE + r * CHUNK, CHUNK)], sg0)
        pltpu.sync_copy(idx_hbm.at[wid], idx_v)
        for r in range(ROW_SPAN // CHUNK):
            pltpu.make_async_copy(
                xbuf0, acc.at[pl.ds(s * ROW_BASE, CHUNK)], sg0).wait()
        plsc.subcore_barrier()

    ebase = wid * E_W

    def gather(j, buf, sem):
        pltpu.async_copy(x_hbm.at[pl.ds(ebase + j * CHUNK, CHUNK)], buf, sem)

    def drain(buf, sem):
        pltpu.make_async_copy(x_hbm.at[pl.ds(ebase, CHUNK)], buf, sem).wait()

    gather(0, xbuf0, sg0)

    def body(p, carry):
        j = 2 * p
        gather(j + 1, xbuf1, sg1)
        drain(xbuf0, sg0)
        pltpu.sync_copy(xbuf0, acc.at[idx_v.at[j]], add=True)
        gather(j + 2, xbuf0, sg0)
        drain(xbuf1, sg1)
        pltpu.sync_copy(xbuf1, acc.at[idx_v.at[j + 1]], add=True)
        return carry

    lax.fori_loop(0, (IDX_ROWS - 1) // 2, body, 0)
    drain(xbuf0, sg0)
    pltpu.sync_copy(xbuf0, acc.at[idx_v.at[IDX_ROWS - 1]], add=True)

    plsc.subcore_barrier()
    pltpu.sync_copy(
        acc.at[pl.ds(s * ROW_BASE, ROW_SPAN)],
        partial_hbm.at[c, pl.ds(s * ROW_BASE, ROW_SPAN)],
    )


def _combine_body(p_ref, o_ref):
    o_ref[...] = p_ref[0] + p_ref[1]


_N_BLK = 10


def _combine(partial):
    return pl.pallas_call(
        _combine_body,
        grid=(_N_BLK,),
        in_specs=[
            pl.BlockSpec((NC, N_NODES // _N_BLK, D), lambda i: (0, i, 0))
        ],
        out_specs=pl.BlockSpec((N_NODES // _N_BLK, D), lambda i: (i, 0)),
        out_shape=jax.ShapeDtypeStruct((N_NODES, D), jnp.float32),
    )(partial)


def kernel(x, index, dim_size):
    idx2d = index.astype(jnp.int32).reshape(NW, IDX_ROWS, CHUNK)
    partial = _seg_sum_sc(x, idx2d)
    return _combine(partial)


# speedup vs baseline: 1.2981x; 1.0481x over previous
import functools

import jax
import jax.numpy as jnp
from jax import lax
from jax.experimental import pallas as pl
from jax.experimental.pallas import tpu as pltpu
from jax.experimental.pallas import tpu_sc as plsc

N_NODES = 10000
N_EDGES = 320000
D = 128
NC, NS = 2, 16
NW = NC * NS
E_W = N_EDGES // NW
CHUNK = 160
N_FULL = E_W // CHUNK
TAIL = E_W - N_FULL * CHUNK
ROW_BASE = 624
ROW_SPAN = 640


@functools.partial(
    pl.kernel,
    out_type=jax.ShapeDtypeStruct((NC, N_NODES, D), jnp.float32),
    mesh=plsc.VectorSubcoreMesh(core_axis_name="c", subcore_axis_name="s"),
    scratch_types=[
        pltpu.VMEM((E_W,), jnp.int32),
        pltpu.VMEM((CHUNK, D), jnp.float32),
        pltpu.VMEM((CHUNK, D), jnp.float32),
        pltpu.VMEM_SHARED((N_NODES, D), jnp.float32),
        pltpu.SemaphoreType.DMA,
        pltpu.SemaphoreType.DMA,
    ],
)
def _seg_sum_sc(x_hbm, idx_hbm, partial_hbm, idx_v, xbuf0, xbuf1,
                acc, sg0, sg1):
    c = lax.axis_index("c")
    s = lax.axis_index("s")
    wid = s * NC + c
    ebase = wid * E_W

    zv = jnp.zeros((16,), jnp.float32)
    for i in range(CHUNK):
        for j in range(D // 16):
            xbuf0[i, pl.ds(j * 16, 16)] = zv
    for r in range(ROW_SPAN // CHUNK):
        pltpu.async_copy(
            xbuf0, acc.at[pl.ds(s * ROW_BASE + r * CHUNK, CHUNK)], sg0)
    pltpu.sync_copy(idx_hbm.at[pl.ds(ebase, E_W)], idx_v)
    for r in range(ROW_SPAN // CHUNK):
        pltpu.make_async_copy(
            xbuf0, acc.at[pl.ds(s * ROW_BASE, CHUNK)], sg0).wait()
    plsc.subcore_barrier()

    def gather(j, buf, sem):
        pltpu.async_copy(x_hbm.at[pl.ds(ebase + j * CHUNK, CHUNK)], buf, sem)

    def drain(buf, sem):
        pltpu.make_async_copy(x_hbm.at[pl.ds(ebase, CHUNK)], buf, sem).wait()

    def scatter(j, buf):
        pltpu.sync_copy(
            buf, acc.at[idx_v.at[pl.ds(j * CHUNK, CHUNK)]], add=True)

    gather(0, xbuf0, sg0)

    def body(p, carry):
        j = 2 * p
        gather(j + 1, xbuf1, sg1)
        drain(xbuf0, sg0)
        scatter(j, xbuf0)
        gather(j + 2, xbuf0, sg0)
        drain(xbuf1, sg1)
        scatter(j + 1, xbuf1)
        return carry

    lax.fori_loop(0, N_FULL // 2 - 1, body, 0)
    gather(N_FULL - 1, xbuf1, sg1)
    drain(xbuf0, sg0)
    scatter(N_FULL - 2, xbuf0)
    pltpu.async_copy(
        x_hbm.at[pl.ds(ebase + N_FULL * CHUNK, TAIL)],
        xbuf0.at[pl.ds(0, TAIL)], sg0)
    drain(xbuf1, sg1)
    scatter(N_FULL - 1, xbuf1)
    pltpu.make_async_copy(
        x_hbm.at[pl.ds(ebase, TAIL)], xbuf0.at[pl.ds(0, TAIL)], sg0).wait()
    pltpu.sync_copy(
        xbuf0.at[pl.ds(0, TAIL)],
        acc.at[idx_v.at[pl.ds(N_FULL * CHUNK, TAIL)]], add=True)

    plsc.subcore_barrier()
    pltpu.sync_copy(
        acc.at[pl.ds(s * ROW_BASE, ROW_SPAN)],
        partial_hbm.at[c, pl.ds(s * ROW_BASE, ROW_SPAN)],
    )


def _combine_body(p_ref, o_ref):
    o_ref[...] = p_ref[0] + p_ref[1]


_N_BLK = 10


def _combine(partial):
    return pl.pallas_call(
        _combine_body,
        grid=(_N_BLK,),
        in_specs=[
            pl.BlockSpec((NC, N_NODES // _N_BLK, D), lambda i: (0, i, 0))
        ],
        out_specs=pl.BlockSpec((N_NODES // _N_BLK, D), lambda i: (i, 0)),
        out_shape=jax.ShapeDtypeStruct((N_NODES, D), jnp.float32),
    )(partial)


def kernel(x, index, dim_size):
    partial = _seg_sum_sc(x, index.astype(jnp.int32))
    return _combine(partial)


# speedup vs baseline: 1.3186x; 1.0157x over previous
import functools

import jax
import jax.numpy as jnp
from jax import lax
from jax.experimental import pallas as pl
from jax.experimental.pallas import tpu as pltpu
from jax.experimental.pallas import tpu_sc as plsc

N_NODES = 10000
N_EDGES = 320000
D = 128
NC, NS = 2, 16
NW = NC * NS
E_W = N_EDGES // NW
CHUNK = 160
N_FULL = E_W // CHUNK
TAIL = E_W - N_FULL * CHUNK
ROW_BASE = 624
ROW_SPAN = 640


@functools.partial(
    pl.kernel,
    out_type=jax.ShapeDtypeStruct((NC, N_NODES, D), jnp.float32),
    mesh=plsc.VectorSubcoreMesh(core_axis_name="c", subcore_axis_name="s"),
    scratch_types=[
        pltpu.VMEM((E_W,), jnp.int32),
        pltpu.VMEM((CHUNK, D), jnp.float32),
        pltpu.VMEM((CHUNK, D), jnp.float32),
        pltpu.VMEM_SHARED((N_NODES, D), jnp.float32),
        pltpu.SemaphoreType.DMA,
        pltpu.SemaphoreType.DMA,
    ],
)
def _seg_sum_sc(x_hbm, idx_hbm, partial_hbm, idx_v, xbuf0, xbuf1,
                acc, sg0, sg1):
    c = lax.axis_index("c")
    s = lax.axis_index("s")
    wid = s * NC + c
    ebase = wid * E_W

    def gather(j, buf, sem):
        pltpu.async_copy(x_hbm.at[pl.ds(ebase + j * CHUNK, CHUNK)], buf, sem)

    def drain(buf, sem):
        pltpu.make_async_copy(x_hbm.at[pl.ds(ebase, CHUNK)], buf, sem).wait()

    def scatter(j, buf):
        pltpu.sync_copy(
            buf, acc.at[idx_v.at[pl.ds(j * CHUNK, CHUNK)]], add=True)

    pltpu.async_copy(idx_hbm.at[pl.ds(ebase, E_W)], idx_v, sg1)
    gather(0, xbuf1, sg1)
    zv = jnp.zeros((16,), jnp.float32)
    for i in range(CHUNK):
        for j in range(D // 16):
            xbuf0[i, pl.ds(j * 16, 16)] = zv
    for r in range(ROW_SPAN // CHUNK):
        pltpu.async_copy(
            xbuf0, acc.at[pl.ds(s * ROW_BASE + r * CHUNK, CHUNK)], sg0)
    for r in range(ROW_SPAN // CHUNK):
        pltpu.make_async_copy(
            xbuf0, acc.at[pl.ds(s * ROW_BASE, CHUNK)], sg0).wait()
    plsc.subcore_barrier()
    pltpu.make_async_copy(
        idx_hbm.at[pl.ds(ebase, E_W)], idx_v, sg1).wait()

    def body(p, carry):
        j = 2 * p
        gather(j + 1, xbuf0, sg0)
        drain(xbuf1, sg1)
        scatter(j, xbuf1)
        gather(j + 2, xbuf1, sg1)
        drain(xbuf0, sg0)
        scatter(j + 1, xbuf0)
        return carry

    lax.fori_loop(0, N_FULL // 2 - 1, body, 0)
    gather(N_FULL - 1, xbuf0, sg0)
    drain(xbuf1, sg1)
    scatter(N_FULL - 2, xbuf1)
    pltpu.async_copy(
        x_hbm.at[pl.ds(ebase + N_FULL * CHUNK, TAIL)],
        xbuf1.at[pl.ds(0, TAIL)], sg1)
    drain(xbuf0, sg0)
    scatter(N_FULL - 1, xbuf0)
    pltpu.make_async_copy(
        x_hbm.at[pl.ds(ebase, TAIL)], xbuf1.at[pl.ds(0, TAIL)], sg1).wait()
    pltpu.sync_copy(
        xbuf1.at[pl.ds(0, TAIL)],
        acc.at[idx_v.at[pl.ds(N_FULL * CHUNK, TAIL)]], add=True)

    plsc.subcore_barrier()
    pltpu.sync_copy(
        acc.at[pl.ds(s * ROW_BASE, ROW_SPAN)],
        partial_hbm.at[c, pl.ds(s * ROW_BASE, ROW_SPAN)],
    )


def _combine_body(p_ref, o_ref):
    o_ref[...] = p_ref[0] + p_ref[1]


_N_BLK = 10


def _combine(partial):
    return pl.pallas_call(
        _combine_body,
        grid=(_N_BLK,),
        in_specs=[
            pl.BlockSpec((NC, N_NODES // _N_BLK, D), lambda i: (0, i, 0))
        ],
        out_specs=pl.BlockSpec((N_NODES // _N_BLK, D), lambda i: (i, 0)),
        out_shape=jax.ShapeDtypeStruct((N_NODES, D), jnp.float32),
    )(partial)


def kernel(x, index, dim_size):
    partial = _seg_sum_sc(x, index.astype(jnp.int32))
    return _combine(partial)


# speedup vs baseline: 1.3347x; 1.0122x over previous
import functools

import jax
import jax.numpy as jnp
from jax import lax
from jax.experimental import pallas as pl
from jax.experimental.pallas import tpu as pltpu
from jax.experimental.pallas import tpu_sc as plsc

N_NODES = 10000
N_EDGES = 320000
D = 128
NC, NS = 2, 16
NW = NC * NS
E_W = N_EDGES // NW
CHUNK = 160
N_FULL = E_W // CHUNK
TAIL = E_W - N_FULL * CHUNK
ROW_BASE = 624
ROW_SPAN = 640


@functools.partial(
    pl.kernel,
    out_type=jax.ShapeDtypeStruct((NC, N_NODES, D), jnp.float32),
    mesh=plsc.VectorSubcoreMesh(core_axis_name="c", subcore_axis_name="s"),
    scratch_types=[
        pltpu.VMEM((E_W,), jnp.int32),
        pltpu.VMEM((CHUNK, D), jnp.float32),
        pltpu.VMEM((CHUNK, D), jnp.float32),
        pltpu.VMEM_SHARED((N_NODES, D), jnp.float32),
        pltpu.SemaphoreType.DMA,
        pltpu.SemaphoreType.DMA,
    ],
)
def _seg_sum_sc(x_hbm, idx_hbm, partial_hbm, idx_v, xbuf0, xbuf1,
                acc, sg0, sg1):
    c = lax.axis_index("c")
    s = lax.axis_index("s")
    wid = s * NC + c
    ebase = wid * E_W

    def gather(j, buf, sem):
        pltpu.async_copy(x_hbm.at[pl.ds(ebase + j * CHUNK, CHUNK)], buf, sem)

    def drain(buf, sem):
        pltpu.make_async_copy(x_hbm.at[pl.ds(ebase, CHUNK)], buf, sem).wait()

    def scatter(j, buf):
        pltpu.sync_copy(
            buf, acc.at[idx_v.at[pl.ds(j * CHUNK, CHUNK)]], add=True)

    pltpu.async_copy(idx_hbm.at[pl.ds(ebase, E_W)], idx_v, sg1)
    gather(0, xbuf1, sg1)
    zv = jnp.zeros((16,), jnp.float32)
    for i in range(CHUNK):
        for j in range(D // 16):
            xbuf0[i, pl.ds(j * 16, 16)] = zv
    for r in range(ROW_SPAN // CHUNK):
        pltpu.async_copy(
            xbuf0, acc.at[pl.ds(s * ROW_BASE + r * CHUNK, CHUNK)], sg0)
    for r in range(ROW_SPAN // CHUNK):
        pltpu.make_async_copy(
            xbuf0, acc.at[pl.ds(s * ROW_BASE, CHUNK)], sg0).wait()
    plsc.subcore_barrier()
    pltpu.make_async_copy(
        idx_hbm.at[pl.ds(ebase, E_W)], idx_v, sg1).wait()

    def body(p, carry):
        j = 2 * p
        gather(j + 1, xbuf0, sg0)
        drain(xbuf1, sg1)
        scatter(j, xbuf1)
        gather(j + 2, xbuf1, sg1)
        drain(xbuf0, sg0)
        scatter(j + 1, xbuf0)
        return carry

    lax.fori_loop(0, N_FULL // 2 - 1, body, 0)
    gather(N_FULL - 1, xbuf0, sg0)
    drain(xbuf1, sg1)
    scatter(N_FULL - 2, xbuf1)
    pltpu.async_copy(
        x_hbm.at[pl.ds(ebase + N_FULL * CHUNK, TAIL)],
        xbuf1.at[pl.ds(0, TAIL)], sg1)
    drain(xbuf0, sg0)
    scatter(N_FULL - 1, xbuf0)
    pltpu.make_async_copy(
        x_hbm.at[pl.ds(ebase, TAIL)], xbuf1.at[pl.ds(0, TAIL)], sg1).wait()
    pltpu.sync_copy(
        xbuf1.at[pl.ds(0, TAIL)],
        acc.at[idx_v.at[pl.ds(N_FULL * CHUNK, TAIL)]], add=True)

    plsc.subcore_barrier()
    pltpu.sync_copy(
        acc.at[pl.ds(s * ROW_BASE, ROW_SPAN)],
        partial_hbm.at[c, pl.ds(s * ROW_BASE, ROW_SPAN)],
    )


def _combine_body(p_ref, o_ref):
    o_ref[...] = p_ref[0] + p_ref[1]


_N_BLK = 5


def _combine(partial):
    return pl.pallas_call(
        _combine_body,
        grid=(_N_BLK,),
        in_specs=[
            pl.BlockSpec((NC, N_NODES // _N_BLK, D), lambda i: (0, i, 0))
        ],
        out_specs=pl.BlockSpec((N_NODES // _N_BLK, D), lambda i: (i, 0)),
        out_shape=jax.ShapeDtypeStruct((N_NODES, D), jnp.float32),
    )(partial)


def kernel(x, index, dim_size):
    partial = _seg_sum_sc(x, index.astype(jnp.int32))
    return _combine(partial)
